# baseline (speedup 1.0000x reference)
import jax
import jax.numpy as jnp
from jax.experimental import pallas as pl
from jax.experimental.pallas import tpu as pltpu
from jax.experimental.pallas import tpu_sc as plsc

D_IN = 768
D_HID = 3072
N_EXP = 16
N_HC = 2
H_BLK = D_HID // N_HC
N_T = N_EXP * N_HC
S = 3
N_TOK = 32


def _logits_kernel(x_ref, WgT_ref, WnT_ref, eps_ref, out_ref):
    xv = x_ref[...]
    gl = jnp.dot(xv, WgT_ref[...], preferred_element_type=jnp.float32)
    nl = jnp.dot(xv, WnT_ref[...], preferred_element_type=jnp.float32)
    out_ref[...] = gl + jax.nn.softplus(nl) * eps_ref[...]


def _make_sc_gating_kernel(n_cores, n_subcores):
    total = n_cores * n_subcores

    def _sc_gating_kernel(l_hbm, w_hbm, vbuf):
        c = jax.lax.axis_index("c")
        s = jax.lax.axis_index("s")
        row0 = c * n_subcores + s

        def do_row(row):
            pltpu.sync_copy(l_hbm.at[row], vbuf)
            l = vbuf[...]
            idx = jax.lax.iota(jnp.int32, 16)
            k, p = plsc.sort_key_val(l, idx, descending=True)
            zeros = jnp.zeros((16,), jnp.int32)
            ones = zeros + 1
            v1 = k.at[zeros].get(mode="promise_in_bounds")
            v2 = k.at[ones].get(mode="promise_in_bounds")
            i1 = p.at[zeros].get(mode="promise_in_bounds")
            i2 = p.at[ones].get(mode="promise_in_bounds")
            t = jnp.exp(v2 - v1)
            denom = 1.0 + t
            w1 = 1.0 / denom
            w2 = t / denom
            wv = (jnp.where(idx == i1, w1, 0.0)
                  + jnp.where(idx == i2, w2, 0.0))
            vbuf[...] = wv
            pltpu.sync_copy(vbuf, w_hbm.at[row])

        for k in range(0, N_TOK, total):
            @pl.when(row0 + k < N_TOK)
            def _(kk=k):
                do_row(row0 + kk)

    return _sc_gating_kernel


def _moe_kernel(x_ref, w_ref, b1_ref, b2_ref,
                W1_hbm, W2_hbm, out_ref, sems, w1b, w2b):
    xv = x_ref[...]

    def w1_copy(t, slot):
        e = t // N_HC
        hc = t % N_HC
        return pltpu.make_async_copy(
            W1_hbm.at[e, :, pl.ds(hc * H_BLK, H_BLK)], w1b.at[slot],
            sems.at[0, slot])

    def w2_copy(t, slot):
        e = t // N_HC
        hc = t % N_HC
        return pltpu.make_async_copy(
            W2_hbm.at[e, pl.ds(hc * H_BLK, H_BLK), :], w2b.at[slot],
            sems.at[1, slot])

    for t0 in range(S):
        w1_copy(t0, t0).start()
        w2_copy(t0, t0).start()

    out_ref[...] = jnp.dot(w_ref[...], b2_ref[...],
                           preferred_element_type=jnp.float32)

    def tick(t, _):
        slot = jax.lax.rem(t, S)
        e = t // N_HC
        hc = jax.lax.rem(t, N_HC)
        w1_copy(t, slot).wait()
        w2_copy(t, slot).wait()
        ei = jax.lax.broadcasted_iota(jnp.int32, (N_TOK, N_EXP), 1)
        w_col = jnp.sum(jnp.where(ei == e, w_ref[...], 0.0), axis=1,
                        keepdims=True)
        h = jnp.dot(xv, w1b[slot], preferred_element_type=jnp.float32)
        h = jnp.maximum(h + b1_ref[pl.ds(e, 1), pl.ds(hc * H_BLK, H_BLK)], 0.0)
        y = jnp.dot(w_col * h, w2b[slot], preferred_element_type=jnp.float32)
        out_ref[...] += y

        @pl.when(t + S < N_T)
        def _refill():
            w1_copy(t + S, slot).start()
            w2_copy(t + S, slot).start()

        return _

    jax.lax.fori_loop(0, N_T, tick, None)


def kernel(x, Wg, Wnoise, W1, b1, W2, b2):
    b, c, d = x.shape
    xm = x.reshape(b * c, d)
    eps = jax.random.normal(jax.random.key(42), (b * c, N_EXP), dtype=x.dtype)

    logits = pl.pallas_call(
        _logits_kernel,
        out_shape=jax.ShapeDtypeStruct((b * c, N_EXP), jnp.float32),
    )(xm, Wg.T, Wnoise.T, eps)

    mesh = plsc.VectorSubcoreMesh(core_axis_name="c", subcore_axis_name="s")
    sc_gate = pl.kernel(
        _make_sc_gating_kernel(mesh.num_cores, mesh.num_subcores),
        out_type=jax.ShapeDtypeStruct((b * c, N_EXP), jnp.float32),
        mesh=mesh,
        scratch_types=[pltpu.VMEM((N_EXP,), jnp.float32)],
        compiler_params=pltpu.CompilerParams(needs_layout_passes=False),
    )
    w = sc_gate(logits)

    out = pl.pallas_call(
        _moe_kernel,
        in_specs=[
            pl.BlockSpec(memory_space=pltpu.VMEM),
            pl.BlockSpec(memory_space=pltpu.VMEM),
            pl.BlockSpec(memory_space=pltpu.VMEM),
            pl.BlockSpec(memory_space=pltpu.VMEM),
            pl.BlockSpec(memory_space=pltpu.HBM),
            pl.BlockSpec(memory_space=pltpu.HBM),
        ],
        out_specs=pl.BlockSpec(memory_space=pltpu.VMEM),
        out_shape=jax.ShapeDtypeStruct((b * c, D_IN), jnp.float32),
        scratch_shapes=[
            pltpu.SemaphoreType.DMA((2, S)),
            pltpu.VMEM((S, D_IN, H_BLK), jnp.float32),
            pltpu.VMEM((S, H_BLK, D_IN), jnp.float32),
        ],
    )(xm, w, b1, b2, W1, W2)
    return out.reshape(b, c, d)


# speedup vs baseline: 1.2658x; 1.2658x over previous
import jax
import jax.numpy as jnp
from jax.experimental import pallas as pl
from jax.experimental.pallas import tpu as pltpu

D_IN = 768
D_HID = 3072
N_EXP = 16
N_HC = 2
H_BLK = D_HID // N_HC
N_T = N_EXP * N_HC


def _moe_kernel(x_ref, Wg_ref, Wn_ref, eps_ref, b1_ref, b2_ref,
                W1_ref, W2_ref, out_ref, w_scr, h_scr):
    t = pl.program_id(0)
    xv = x_ref[...]

    @pl.when(t == 0)
    def _init():
        gl = jnp.dot(xv, Wg_ref[...].T, preferred_element_type=jnp.float32)
        nl = jnp.dot(xv, Wn_ref[...].T, preferred_element_type=jnp.float32)
        logits = gl + jax.nn.softplus(nl) * eps_ref[...]
        eidx = jax.lax.broadcasted_iota(jnp.int32, logits.shape, 1)
        v1 = jnp.max(logits, axis=-1, keepdims=True)
        i1 = jnp.argmax(logits, axis=-1)[:, None]
        masked = jnp.where(eidx == i1, -jnp.inf, logits)
        i2 = jnp.argmax(masked, axis=-1)[:, None]
        sel = (eidx == i1) | (eidx == i2)
        ew = jnp.where(sel, jnp.exp(logits - v1), 0.0)
        w = ew / jnp.sum(ew, axis=-1, keepdims=True)
        w_scr[...] = w
        out_ref[...] = jnp.dot(w, b2_ref[...], preferred_element_type=jnp.float32)

    p = jax.lax.rem(t, 2)

    @pl.when(t < N_T)
    def _first_layer():
        h_scr[p] = jnp.dot(xv, W1_ref[0], preferred_element_type=jnp.float32)

    @pl.when(t >= 1)
    def _second_layer():
        q = t - 1
        qe = q // N_HC
        qh = jax.lax.rem(q, N_HC)
        h = jnp.maximum(h_scr[1 - p]
                        + b1_ref[pl.ds(qe, 1), pl.ds(qh * H_BLK, H_BLK)], 0.0)
        eidx = jax.lax.broadcasted_iota(jnp.int32, (32, N_EXP), 1)
        w_col = jnp.sum(jnp.where(eidx == qe, w_scr[...], 0.0), axis=1,
                        keepdims=True)
        out_ref[...] += jnp.dot(w_col * h, W2_ref[0],
                                preferred_element_type=jnp.float32)


def kernel(x, Wg, Wnoise, W1, b1, W2, b2):
    b, c, d = x.shape
    xm = x.reshape(b * c, d)
    eps = jax.random.normal(jax.random.key(42), (b * c, N_EXP), dtype=x.dtype)

    def w1_idx(t):
        tc = jnp.where(t < N_T, t, N_T - 1)
        return (tc // N_HC, 0, jax.lax.rem(tc, N_HC))

    def w2_idx(t):
        q = jnp.where(t >= 1, t - 1, 0)
        return (q // N_HC, jax.lax.rem(q, N_HC), 0)

    out = pl.pallas_call(
        _moe_kernel,
        grid=(N_T + 1,),
        in_specs=[
            pl.BlockSpec((b * c, D_IN), lambda t: (0, 0)),
            pl.BlockSpec((N_EXP, D_IN), lambda t: (0, 0)),
            pl.BlockSpec((N_EXP, D_IN), lambda t: (0, 0)),
            pl.BlockSpec((b * c, N_EXP), lambda t: (0, 0)),
            pl.BlockSpec((N_EXP, D_HID), lambda t: (0, 0)),
            pl.BlockSpec((N_EXP, D_IN), lambda t: (0, 0)),
            pl.BlockSpec((1, D_IN, H_BLK), w1_idx),
            pl.BlockSpec((1, H_BLK, D_IN), w2_idx),
        ],
        out_specs=pl.BlockSpec((b * c, D_IN), lambda t: (0, 0)),
        out_shape=jax.ShapeDtypeStruct((b * c, D_IN), jnp.float32),
        scratch_shapes=[pltpu.VMEM((b * c, N_EXP), jnp.float32),
                        pltpu.VMEM((2, b * c, H_BLK), jnp.float32)],
    )(xm, Wg, Wnoise, eps, b1, b2, W1, W2)
    return out.reshape(b, c, d)


# speedup vs baseline: 1.2717x; 1.0047x over previous
import jax
import jax.numpy as jnp
from jax.experimental import pallas as pl
from jax.experimental.pallas import tpu as pltpu

D_IN = 768
D_HID = 3072
N_EXP = 16
N_HC = 2
H_BLK = D_HID // N_HC


def _moe_kernel(x_ref, Wg_ref, Wn_ref, eps_ref, b1_ref, b2_ref,
                W1_ref, W2_ref, out_ref, w_scr):
    e = pl.program_id(0)
    hc = pl.program_id(1)
    xv = x_ref[...]

    @pl.when((e == 0) & (hc == 0))
    def _init():
        gl = jnp.dot(xv, Wg_ref[...].T, preferred_element_type=jnp.float32)
        nl = jnp.dot(xv, Wn_ref[...].T, preferred_element_type=jnp.float32)
        logits = gl + jax.nn.softplus(nl) * eps_ref[...]
        eidx = jax.lax.broadcasted_iota(jnp.int32, logits.shape, 1)
        v1 = jnp.max(logits, axis=-1, keepdims=True)
        i1 = jnp.argmax(logits, axis=-1)[:, None]
        masked = jnp.where(eidx == i1, -jnp.inf, logits)
        i2 = jnp.argmax(masked, axis=-1)[:, None]
        sel = (eidx == i1) | (eidx == i2)
        ew = jnp.where(sel, jnp.exp(logits - v1), 0.0)
        w = ew / jnp.sum(ew, axis=-1, keepdims=True)
        w_scr[...] = w
        out_ref[...] = jnp.dot(w, b2_ref[...], preferred_element_type=jnp.float32)

    eidx = jax.lax.broadcasted_iota(jnp.int32, (32, N_EXP), 1)
    w_col = jnp.sum(jnp.where(eidx == e, w_scr[...], 0.0), axis=1, keepdims=True)
    h = jnp.dot(xv, W1_ref[0], preferred_element_type=jnp.float32)
    h = jnp.maximum(h + b1_ref[pl.ds(e, 1), pl.ds(hc * H_BLK, H_BLK)], 0.0)
    out_ref[...] += jnp.dot(w_col * h, W2_ref[0],
                            preferred_element_type=jnp.float32)


def kernel(x, Wg, Wnoise, W1, b1, W2, b2):
    b, c, d = x.shape
    xm = x.reshape(b * c, d)
    eps = jax.random.normal(jax.random.key(42), (b * c, N_EXP), dtype=x.dtype)

    out = pl.pallas_call(
        _moe_kernel,
        grid=(N_EXP, N_HC),
        in_specs=[
            pl.BlockSpec((b * c, D_IN), lambda e, hc: (0, 0)),
            pl.BlockSpec((N_EXP, D_IN), lambda e, hc: (0, 0)),
            pl.BlockSpec((N_EXP, D_IN), lambda e, hc: (0, 0)),
            pl.BlockSpec((b * c, N_EXP), lambda e, hc: (0, 0)),
            pl.BlockSpec((N_EXP, D_HID), lambda e, hc: (0, 0)),
            pl.BlockSpec((N_EXP, D_IN), lambda e, hc: (0, 0)),
            pl.BlockSpec((1, D_IN, H_BLK), lambda e, hc: (e, 0, hc)),
            pl.BlockSpec((1, H_BLK, D_IN), lambda e, hc: (e, hc, 0)),
        ],
        out_specs=pl.BlockSpec((b * c, D_IN), lambda e, hc: (0, 0)),
        out_shape=jax.ShapeDtypeStruct((b * c, D_IN), jnp.float32),
        scratch_shapes=[pltpu.VMEM((b * c, N_EXP), jnp.float32)],
    )(xm, Wg, Wnoise, eps, b1, b2, W1, W2)
    return out.reshape(b, c, d)
